# baseline (speedup 1.0000x reference)
import functools

import jax
import jax.numpy as jnp
from jax import lax
from jax.experimental import pallas as pl
from jax.experimental.pallas import tpu as pltpu
from jax.experimental.pallas import tpu_sc as plsc

N = 10000
D = 128
DP = D // 2
NC = 2
NS = 16
NW = NC * NS
WE = 128
NACC = 10112
PTN = NACC // NS
NBUF = 2
NIB = 4
NIS = 3

_mesh = plsc.VectorSubcoreMesh(core_axis_name="c", subcore_axis_name="s")


DEGN = 10240
DEGP = DEGN // NS


def _make_deg(nrow):
    rw, rem = nrow // NW, nrow % NW

    @functools.partial(
        pl.kernel,
        out_type=jax.ShapeDtypeStruct((NC, DEGN), jnp.float32),
        mesh=_mesh,
        scratch_types=[
            pltpu.VMEM((NIB, WE), jnp.int32),
            pltpu.VMEM((DEGP,), jnp.float32),
            pltpu.VMEM((WE,), jnp.float32),
            pltpu.VMEM_SHARED((DEGN,), jnp.float32),
            [pltpu.SemaphoreType.DMA] * NIB,
        ],
    )
    def deg_kernel(dst_hbm, deg_out, dring, zv, ones_v, acc, isems):
        c = lax.axis_index("c")
        s = lax.axis_index("s")
        w = c * NS + s
        r0 = w * rw

        def fz(i, _):
            zv[pl.ds(i * 16, 16)] = jnp.zeros((16,), jnp.float32)
            return _
        lax.fori_loop(0, DEGP // 16, fz, None)

        def fo(i, _):
            ones_v[pl.ds(i * 16, 16)] = jnp.ones((16,), jnp.float32)
            return _
        lax.fori_loop(0, WE // 16, fo, None)

        def i_start(j):
            ib = j % NIB
            pltpu.async_copy(dst_hbm.at[pl.ds((r0 + j) * WE, WE)],
                             dring.at[ib], isems[ib])

        def i_wait(j):
            ib = j % NIB
            pltpu.make_async_copy(dst_hbm.at[pl.ds((r0 + j) * WE, WE)],
                                  dring.at[ib], isems[ib]).wait()

        pltpu.sync_copy(zv, acc.at[pl.ds(s * DEGP, DEGP)])
        for j in range(min(3, rw)):
            i_start(j)
        plsc.subcore_barrier()

        for j in range(rw):
            i_wait(j)
            if j + 3 < rw:
                i_start(j + 3)
            pltpu.sync_copy(ones_v, acc.at[dring.at[j % NIB]], add=True)
        if rem:
            @pl.when(w < rem)
            def _():
                pltpu.sync_copy(
                    dst_hbm.at[pl.ds((NW * rw + w) * WE, WE)], dring.at[0])
                pltpu.sync_copy(ones_v, acc.at[dring.at[0]], add=True)

        plsc.subcore_barrier()
        pltpu.sync_copy(acc.at[pl.ds(s * DEGP, DEGP)],
                        deg_out.at[c, pl.ds(s * DEGP, DEGP)])

    return deg_kernel


def _make_agg(nrow):
    rw, rem = nrow // NW, nrow % NW
    assert rw >= 2 * NIB

    @functools.partial(
        pl.kernel,
        out_type=jax.ShapeDtypeStruct((NC, NACC, D), jnp.float32),
        mesh=_mesh,
        scratch_types=[
            pltpu.VMEM((NIS, WE), jnp.int32),
            pltpu.VMEM((NIB, WE), jnp.int32),
            pltpu.VMEM((NBUF * WE, DP), jnp.int32),
            pltpu.VMEM((NBUF * WE, D), jnp.float32),
            pltpu.VMEM_SHARED((NACC, D), jnp.float32),
            [pltpu.SemaphoreType.DMA] * NIB,
            [pltpu.SemaphoreType.DMA] * NBUF,
            [pltpu.SemaphoreType.DMA] * NBUF,
        ],
        compiler_params=pltpu.CompilerParams(use_tc_tiling_on_sc=False),
    )
    def agg_kernel(m_hbm, src_hbm, dst_hbm, g_out,
                   sring, dring, gbuf, sbuf, acc, isems, gsems, ssems):
        c = lax.axis_index("c")
        s = lax.axis_index("s")
        w = c * NS + s
        r0 = w * rw

        def fz(i, _):
            sbuf[i // 8, pl.ds((i % 8) * 16, 16)] = (
                jnp.zeros((16,), jnp.float32))
            return _
        lax.fori_loop(0, WE * 8, fz, None)

        base = s * PTN
        for k in range(PTN // WE):
            pltpu.sync_copy(sbuf.at[pl.ds(0, WE)],
                            acc.at[pl.ds(base + k * WE, WE)])
        ztail = PTN % WE
        if ztail:
            pltpu.sync_copy(sbuf.at[pl.ds(0, ztail)],
                            acc.at[pl.ds(base + (PTN // WE) * WE, ztail)])
        plsc.subcore_barrier()

        def i_start(j):
            pltpu.async_copy(src_hbm.at[pl.ds((r0 + j) * WE, WE)],
                             sring.at[j % NIS], isems[j % NIB])
            pltpu.async_copy(dst_hbm.at[pl.ds((r0 + j) * WE, WE)],
                             dring.at[j % NIB], isems[j % NIB])

        def i_wait(j):
            pltpu.make_async_copy(src_hbm.at[pl.ds((r0 + j) * WE, WE)],
                                  sring.at[j % NIS], isems[j % NIB]).wait()
            pltpu.make_async_copy(dst_hbm.at[pl.ds((r0 + j) * WE, WE)],
                                  dring.at[j % NIB], isems[j % NIB]).wait()

        def g_start(j):
            b = j % NBUF
            pltpu.async_copy(m_hbm.at[sring.at[j % NIS]],
                             gbuf.at[pl.ds(b * WE, WE)], gsems[b])

        def g_wait(j):
            b = j % NBUF
            pltpu.make_async_copy(m_hbm.at[sring.at[j % NIS]],
                                  gbuf.at[pl.ds(b * WE, WE)],
                                  gsems[b]).wait()

        def s_start(j):
            b = j % NBUF
            pltpu.async_copy(sbuf.at[pl.ds(b * WE, WE)],
                             acc.at[dring.at[j % NIB]], ssems[b], add=True)

        def s_wait(j):
            b = j % NBUF
            pltpu.make_async_copy(sbuf.at[pl.ds(b * WE, WE)],
                                  acc.at[dring.at[j % NIB]],
                                  ssems[b]).wait()

        def convert(j):
            b = j % NBUF
            hmask = jnp.full((16,), -65536, dtype=jnp.int32)

            def rows(i, _):
                r = b * WE + i * 2
                for rr in range(2):
                    for cch in range(DP // 16):
                        v = gbuf[r + rr, pl.ds(cch * 16, 16)]
                        sbuf[r + rr, pl.ds(cch * 16, 16)] = (
                            lax.bitcast_convert_type(
                                jnp.left_shift(v, 16), jnp.float32))
                        sbuf[r + rr, pl.ds(DP + cch * 16, 16)] = (
                            lax.bitcast_convert_type(
                                v & hmask, jnp.float32))
                return _
            lax.fori_loop(0, WE // 2, rows, None)

        i_start(0)
        i_start(1)
        i_wait(0)
        g_start(0)
        for j in range(rw):
            if j >= 2:
                s_wait(j - 2)
            if j + 1 < rw:
                i_wait(j + 1)
                g_start(j + 1)
            if j + 2 < rw:
                i_start(j + 2)
            g_wait(j)
            convert(j)
            s_start(j)
        s_wait(rw - 2)
        s_wait(rw - 1)

        if rem:
            @pl.when(w < rem)
            def _():
                roff = (NW * rw + w) * WE
                pltpu.sync_copy(src_hbm.at[pl.ds(roff, WE)], sring.at[0])
                pltpu.sync_copy(dst_hbm.at[pl.ds(roff, WE)], dring.at[0])
                pltpu.sync_copy(m_hbm.at[sring.at[0]],
                                gbuf.at[pl.ds(0, WE)])
                convert(0)
                pltpu.sync_copy(sbuf.at[pl.ds(0, WE)], acc.at[dring.at[0]],
                                add=True)

        plsc.subcore_barrier()
        pltpu.sync_copy(acc.at[pl.ds(base, PTN)],
                        g_out.at[c, pl.ds(base, PTN)])

    return agg_kernel


BR = 2000


def _pack(h):
    hb = h.astype(jnp.bfloat16).astype(jnp.float32)
    lo = lax.bitcast_convert_type(hb[:, :DP], jnp.uint32) >> 16
    hi = lax.bitcast_convert_type(hb[:, DP:], jnp.uint32) & jnp.uint32(
        0xFFFF0000)
    return lax.bitcast_convert_type(hi | lo, jnp.int32)


def _prep_body(x_ref, w_ref, deg_ref, m_ref, dinv_ref):
    deg = deg_ref[0] + deg_ref[1]
    dv = jnp.where(deg > 0.0, lax.rsqrt(jnp.maximum(deg, 1e-12)), 0.0)
    dinv_ref[...] = dv
    m_ref[...] = _pack(jnp.dot(x_ref[...], w_ref[...]) * dv)


def _prep(x, W1, deg2):
    grid = N // BR
    return pl.pallas_call(
        _prep_body,
        grid=(grid,),
        in_specs=[
            pl.BlockSpec((BR, D), lambda i: (i, 0)),
            pl.BlockSpec((D, D), lambda i: (0, 0)),
            pl.BlockSpec((NC, BR, 1), lambda i: (0, i, 0)),
        ],
        out_specs=[
            pl.BlockSpec((BR, DP), lambda i: (i, 0)),
            pl.BlockSpec((BR, 1), lambda i: (i, 0)),
        ],
        out_shape=[
            jax.ShapeDtypeStruct((N, DP), jnp.int32),
            jax.ShapeDtypeStruct((N, 1), jnp.float32),
        ],
    )(x, W1, deg2)


def _mid_body(g_ref, dinv_ref, b_ref, w_ref, m_ref):
    dv = dinv_ref[...]
    agg = (g_ref[0] + g_ref[1]) * dv + b_ref[...]
    z = jnp.maximum(agg, 0.0)
    m_ref[...] = _pack(jnp.dot(z, w_ref[...]) * dv)


def _mid(g, dinv, b, W):
    grid = N // BR
    return pl.pallas_call(
        _mid_body,
        grid=(grid,),
        in_specs=[
            pl.BlockSpec((NC, BR, D), lambda i: (0, i, 0)),
            pl.BlockSpec((BR, 1), lambda i: (i, 0)),
            pl.BlockSpec((1, D), lambda i: (0, 0)),
            pl.BlockSpec((D, D), lambda i: (0, 0)),
        ],
        out_specs=pl.BlockSpec((BR, DP), lambda i: (i, 0)),
        out_shape=jax.ShapeDtypeStruct((N, DP), jnp.int32),
    )(g, dinv, b.reshape(1, D), W)


def _final_body(g_ref, dinv_ref, b_ref, o_ref):
    dv = dinv_ref[...]
    o_ref[...] = (g_ref[0] + g_ref[1]) * dv + b_ref[...]


def _final(g, dinv, b):
    grid = N // BR
    return pl.pallas_call(
        _final_body,
        grid=(grid,),
        in_specs=[
            pl.BlockSpec((NC, BR, D), lambda i: (0, i, 0)),
            pl.BlockSpec((BR, 1), lambda i: (i, 0)),
            pl.BlockSpec((1, D), lambda i: (0, 0)),
        ],
        out_specs=pl.BlockSpec((BR, D), lambda i: (i, 0)),
        out_shape=jax.ShapeDtypeStruct((N, D), jnp.float32),
    )(g, dinv, b.reshape(1, D))


def kernel(x, adj_t, W1, b1, W2, b2, W3, b3):
    adj = adj_t.astype(jnp.int32)
    E = adj.shape[1]
    src, dst = adj[0], adj[1]
    tail = (-E) % WE
    if tail:
        prange = jnp.arange(tail, dtype=jnp.int32)
        src = jnp.concatenate([src, prange % N])
        dst = jnp.concatenate([dst, N + prange % (NACC - N)])
    nrow = (E + tail) // WE

    deg2 = _make_deg(nrow)(dst)
    agg = _make_agg(nrow)
    m1, dinv = _prep(x, W1, deg2.reshape(NC, DEGN, 1))
    g1 = agg(m1, src, dst)
    m2 = _mid(g1, dinv, b1, W2)
    g2 = agg(m2, src, dst)
    m3 = _mid(g2, dinv, b2, W3)
    g3 = agg(m3, src, dst)
    return _final(g3, dinv, b3)


# speedup vs baseline: 1.8336x; 1.8336x over previous
import functools

import jax
import jax.numpy as jnp
from jax import lax
from jax.experimental import pallas as pl
from jax.experimental.pallas import tpu as pltpu
from jax.experimental.pallas import tpu_sc as plsc

N = 10000
D = 128
NC = 2
NS = 16
NW = NC * NS
WE = 128
NACC = 10240
PTN = NACC // NS
NBUF = 2
NIB = 4

_mesh = plsc.VectorSubcoreMesh(core_axis_name="c", subcore_axis_name="s")


def _make_deg(nrow):
    rw, rem = nrow // NW, nrow % NW

    @functools.partial(
        pl.kernel,
        out_type=jax.ShapeDtypeStruct((NC, NACC), jnp.float32),
        mesh=_mesh,
        scratch_types=[
            pltpu.VMEM((NIB, WE), jnp.int32),
            pltpu.VMEM((PTN,), jnp.float32),
            pltpu.VMEM((WE,), jnp.float32),
            pltpu.VMEM_SHARED((NACC,), jnp.float32),
            [pltpu.SemaphoreType.DMA] * NIB,
        ],
    )
    def deg_kernel(dst_hbm, deg_out, dring, zv, ones_v, acc, isems):
        c = lax.axis_index("c")
        s = lax.axis_index("s")
        w = c * NS + s
        r0 = w * rw

        def fz(i, _):
            zv[pl.ds(i * 16, 16)] = jnp.zeros((16,), jnp.float32)
            return _
        lax.fori_loop(0, PTN // 16, fz, None)

        def fo(i, _):
            ones_v[pl.ds(i * 16, 16)] = jnp.ones((16,), jnp.float32)
            return _
        lax.fori_loop(0, WE // 16, fo, None)

        def i_start(j):
            ib = j % NIB
            pltpu.async_copy(dst_hbm.at[pl.ds((r0 + j) * WE, WE)],
                             dring.at[ib], isems[ib])

        def i_wait(j):
            ib = j % NIB
            pltpu.make_async_copy(dst_hbm.at[pl.ds((r0 + j) * WE, WE)],
                                  dring.at[ib], isems[ib]).wait()

        pltpu.sync_copy(zv, acc.at[pl.ds(s * PTN, PTN)])
        for j in range(min(3, rw)):
            i_start(j)
        plsc.subcore_barrier()

        for j in range(rw):
            i_wait(j)
            if j + 3 < rw:
                i_start(j + 3)
            pltpu.sync_copy(ones_v, acc.at[dring.at[j % NIB]], add=True)
        if rem:
            @pl.when(w < rem)
            def _():
                pltpu.sync_copy(
                    dst_hbm.at[pl.ds((NW * rw + w) * WE, WE)], dring.at[0])
                pltpu.sync_copy(ones_v, acc.at[dring.at[0]], add=True)

        plsc.subcore_barrier()
        pltpu.sync_copy(acc.at[pl.ds(s * PTN, PTN)],
                        deg_out.at[c, pl.ds(s * PTN, PTN)])

    return deg_kernel


def _make_agg(nrow):
    rw, rem = nrow // NW, nrow % NW
    assert rw >= 2 * NIB

    @functools.partial(
        pl.kernel,
        out_type=jax.ShapeDtypeStruct((NC, NACC, D), jnp.float32),
        mesh=_mesh,
        scratch_types=[
            pltpu.VMEM((NIB, WE), jnp.int32),
            pltpu.VMEM((NIB, WE), jnp.int32),
            pltpu.VMEM((NBUF, WE, D), jnp.float32),
            pltpu.VMEM((16, D), jnp.float32),
            pltpu.VMEM_SHARED((NACC, D), jnp.float32),
            [pltpu.SemaphoreType.DMA] * NIB,
            [pltpu.SemaphoreType.DMA] * NBUF,
            [pltpu.SemaphoreType.DMA] * NBUF,
        ],
    )
    def agg_kernel(m_hbm, src_hbm, dst_hbm, g_out,
                   sring, dring, buf, zb, acc, isems, gsems, ssems):
        c = lax.axis_index("c")
        s = lax.axis_index("s")
        w = c * NS + s
        r0 = w * rw

        def fz(i, _):
            zb[i // 8, pl.ds((i % 8) * 16, 16)] = jnp.zeros((16,), jnp.float32)
            return _
        lax.fori_loop(0, 16 * 8, fz, None)

        base = s * PTN

        def zacc(k, _):
            pltpu.sync_copy(zb, acc.at[pl.ds(base + k * 16, 16)])
            return _
        lax.fori_loop(0, PTN // 16, zacc, None)
        plsc.subcore_barrier()

        def i_start(j):
            ib = j % NIB
            pltpu.async_copy(src_hbm.at[pl.ds((r0 + j) * WE, WE)],
                             sring.at[ib], isems[ib])
            pltpu.async_copy(dst_hbm.at[pl.ds((r0 + j) * WE, WE)],
                             dring.at[ib], isems[ib])

        def i_wait(j):
            ib = j % NIB
            pltpu.make_async_copy(src_hbm.at[pl.ds((r0 + j) * WE, WE)],
                                  sring.at[ib], isems[ib]).wait()
            pltpu.make_async_copy(dst_hbm.at[pl.ds((r0 + j) * WE, WE)],
                                  dring.at[ib], isems[ib]).wait()

        def g_start(j):
            b = j % NBUF
            pltpu.async_copy(m_hbm.at[sring.at[j % NIB]], buf.at[b], gsems[b])

        def g_wait(j):
            b = j % NBUF
            pltpu.make_async_copy(m_hbm.at[sring.at[j % NIB]], buf.at[b],
                                  gsems[b]).wait()

        def s_start(j):
            b = j % NBUF
            pltpu.async_copy(buf.at[b], acc.at[dring.at[j % NIB]],
                             ssems[b], add=True)

        def s_wait(j):
            b = j % NBUF
            pltpu.make_async_copy(buf.at[b], acc.at[dring.at[j % NIB]],
                                  ssems[b]).wait()

        i_start(0)
        i_start(1)
        i_wait(0)
        g_start(0)
        for j in range(rw):
            if j >= 1:
                s_wait(j - 1)
            if j + 1 < rw:
                i_wait(j + 1)
                g_start(j + 1)
            if j + 2 < rw:
                i_start(j + 2)
            g_wait(j)
            s_start(j)
        s_wait(rw - 1)

        if rem:
            @pl.when(w < rem)
            def _():
                roff = (NW * rw + w) * WE
                pltpu.sync_copy(src_hbm.at[pl.ds(roff, WE)], sring.at[0])
                pltpu.sync_copy(dst_hbm.at[pl.ds(roff, WE)], dring.at[0])
                pltpu.sync_copy(m_hbm.at[sring.at[0]], buf.at[0])
                pltpu.sync_copy(buf.at[0], acc.at[dring.at[0]], add=True)

        plsc.subcore_barrier()
        pltpu.sync_copy(acc.at[pl.ds(base, PTN)],
                        g_out.at[c, pl.ds(base, PTN)])

    return agg_kernel


BR = 2000


def _prep_body(x_ref, w_ref, deg_ref, m_ref, dinv_ref):
    deg = deg_ref[0] + deg_ref[1]
    dv = jnp.where(deg > 0.0, lax.rsqrt(jnp.maximum(deg, 1e-12)), 0.0)
    dinv_ref[...] = dv
    m_ref[...] = jnp.dot(x_ref[...], w_ref[...]) * dv


def _prep(x, W1, deg2):
    grid = N // BR
    return pl.pallas_call(
        _prep_body,
        grid=(grid,),
        in_specs=[
            pl.BlockSpec((BR, D), lambda i: (i, 0)),
            pl.BlockSpec((D, D), lambda i: (0, 0)),
            pl.BlockSpec((NC, BR, 1), lambda i: (0, i, 0)),
        ],
        out_specs=[
            pl.BlockSpec((BR, D), lambda i: (i, 0)),
            pl.BlockSpec((BR, 1), lambda i: (i, 0)),
        ],
        out_shape=[
            jax.ShapeDtypeStruct((N, D), jnp.float32),
            jax.ShapeDtypeStruct((N, 1), jnp.float32),
        ],
    )(x, W1, deg2)


def _mid_body(g_ref, dinv_ref, b_ref, w_ref, m_ref):
    dv = dinv_ref[...]
    agg = (g_ref[0] + g_ref[1]) * dv + b_ref[...]
    z = jnp.maximum(agg, 0.0)
    m_ref[...] = jnp.dot(z, w_ref[...]) * dv


def _mid(g, dinv, b, W):
    grid = N // BR
    return pl.pallas_call(
        _mid_body,
        grid=(grid,),
        in_specs=[
            pl.BlockSpec((NC, BR, D), lambda i: (0, i, 0)),
            pl.BlockSpec((BR, 1), lambda i: (i, 0)),
            pl.BlockSpec((1, D), lambda i: (0, 0)),
            pl.BlockSpec((D, D), lambda i: (0, 0)),
        ],
        out_specs=pl.BlockSpec((BR, D), lambda i: (i, 0)),
        out_shape=jax.ShapeDtypeStruct((N, D), jnp.float32),
    )(g, dinv, b.reshape(1, D), W)


def _final_body(g_ref, dinv_ref, b_ref, o_ref):
    dv = dinv_ref[...]
    o_ref[...] = (g_ref[0] + g_ref[1]) * dv + b_ref[...]


def _final(g, dinv, b):
    grid = N // BR
    return pl.pallas_call(
        _final_body,
        grid=(grid,),
        in_specs=[
            pl.BlockSpec((NC, BR, D), lambda i: (0, i, 0)),
            pl.BlockSpec((BR, 1), lambda i: (i, 0)),
            pl.BlockSpec((1, D), lambda i: (0, 0)),
        ],
        out_specs=pl.BlockSpec((BR, D), lambda i: (i, 0)),
        out_shape=jax.ShapeDtypeStruct((N, D), jnp.float32),
    )(g, dinv, b.reshape(1, D))


def kernel(x, adj_t, W1, b1, W2, b2, W3, b3):
    adj = adj_t.astype(jnp.int32)
    E = adj.shape[1]
    src, dst = adj[0], adj[1]
    tail = (-E) % WE
    if tail:
        prange = jnp.arange(tail, dtype=jnp.int32)
        src = jnp.concatenate([src, prange % N])
        dst = jnp.concatenate([dst, N + prange % (NACC - N)])
    nrow = (E + tail) // WE

    deg2 = _make_deg(nrow)(dst)
    agg = _make_agg(nrow)
    m1, dinv = _prep(x, W1, deg2.reshape(NC, NACC, 1))
    g1 = agg(m1, src, dst)
    m2 = _mid(g1, dinv, b1, W2)
    g2 = agg(m2, src, dst)
    m3 = _mid(g2, dinv, b2, W3)
    g3 = agg(m3, src, dst)
    return _final(g3, dinv, b3)
